# baseline (speedup 1.0000x reference)
import functools

import jax
import jax.numpy as jnp
from jax.experimental import pallas as pl
from jax.experimental.pallas import tpu as pltpu

_HIGH = jax.lax.Precision.HIGHEST
_PAR = pltpu.CompilerParams(dimension_semantics=("parallel",))

_NPOINTS = [1024, 512, 256, 128]
_RADII = [0.05, 0.1, 0.2, 0.3]
_NSAMPLE = 32


def _iota(shape, dim):
    return jax.lax.broadcasted_iota(jnp.int32, shape, dim)


def _fps_level(X, Y, Z, npoint):
    b, n = X.shape
    lane_n = _iota((b, n), 1)
    lane_n3 = _iota((3 * b, n), 1)
    lane_p = _iota((b, npoint), 1)
    S = jnp.concatenate([X, Y, Z], axis=0)

    def body(i, st):
        dists, far, nx, ny, nz = st
        far3 = jnp.concatenate([far, far, far], axis=0)
        csum = jnp.sum(jnp.where(lane_n3 == far3, S, 0.0), axis=1,
                       keepdims=True)
        cx, cy, cz = csum[:b], csum[b:2 * b], csum[2 * b:]
        m = lane_p == i
        nx = jnp.where(m, cx, nx)
        ny = jnp.where(m, cy, ny)
        nz = jnp.where(m, cz, nz)
        dx = X - cx
        dy = Y - cy
        dz = Z - cz
        d = (dx * dx + dz * dz) + dy * dy
        dists = jnp.minimum(dists, d)
        mx = jnp.max(dists, axis=1, keepdims=True)
        far = jnp.min(jnp.where(dists == mx, lane_n, n), axis=1, keepdims=True)
        return (dists, far, nx, ny, nz)

    init = (
        jnp.full((b, n), 1e10, jnp.float32),
        jnp.zeros((b, 1), jnp.int32),
        jnp.zeros((b, npoint), jnp.float32),
        jnp.zeros((b, npoint), jnp.float32),
        jnp.zeros((b, npoint), jnp.float32),
    )
    _, _, nx, ny, nz = jax.lax.fori_loop(0, npoint, body, init)
    return nx, ny, nz


def _fps_body(x_ref, y_ref, z_ref, *out_refs):
    X, Y, Z = x_ref[...], y_ref[...], z_ref[...]
    for lvl, npoint in enumerate(_NPOINTS):
        X, Y, Z = _fps_level(X, Y, Z, npoint)
        out_refs[3 * lvl][...] = X
        out_refs[3 * lvl + 1][...] = Y
        out_refs[3 * lvl + 2][...] = Z


def _run_fps(x0, y0, z0):
    b = x0.shape[0]
    outs = []
    for npoint in _NPOINTS:
        outs += [jax.ShapeDtypeStruct((b, npoint), jnp.float32)] * 3
    return pl.pallas_call(_fps_body, out_shape=outs)(x0, y0, z0)


def _prefix_sum_lanes(x):
    s, n = x.shape
    sh = 1
    while sh < n:
        pad = jnp.zeros((s, sh), x.dtype)
        x = x + jnp.concatenate([pad, x[:, : n - sh]], axis=1)
        sh *= 2
    return x


def _sa_body(src_ref, xt_ref, cen_ref, w1_ref, b1_ref, w2_ref, b2_ref,
             w3_ref, b3_ref, out_ref, *, r2, nsample):
    src = src_ref[0]
    xt = xt_ref[0]
    cen = cen_ref[0]
    n = src.shape[0]
    s = cen.shape[0]
    C = src.shape[1]

    xr, yr, zr = xt[0:1], xt[1:2], xt[2:3]
    sum_b = (xr * xr + yr * yr) + zr * zr
    ca, cb, cc = cen[:, 0:1], cen[:, 1:2], cen[:, 2:3]
    sum_a = (ca * ca + cb * cb) + cc * cc
    cross = jnp.dot(cen, xt)
    sqd = (sum_a + sum_b) - 2.0 * cross

    valid = sqd <= r2
    rank = _prefix_sum_lanes(valid.astype(jnp.int32))
    count = rank[:, n - 1:n]
    rankv = jnp.where(valid, rank, 0)
    padcen = jnp.concatenate(
        [cen, jnp.zeros((s, C - 3), jnp.float32)], axis=1)

    w1, b1 = w1_ref[...], b1_ref[...]
    w2, b2 = w2_ref[...], b2_ref[...]
    w3, b3 = w3_ref[...], b3_ref[...]

    def mlp(x):
        h = jnp.maximum(jnp.dot(x, w1) + b1, 0.0)
        h = jnp.maximum(jnp.dot(h, w2) + b2, 0.0)
        return jnp.maximum(jnp.dot(h, w3) + b3, 0.0)

    src_hi = src.astype(jnp.bfloat16)
    src_lo = (src - src_hi.astype(jnp.float32)).astype(jnp.bfloat16)

    def gather(sel_bool):
        sel = sel_bool.astype(jnp.bfloat16)
        return (jax.lax.dot(sel, src_hi, preferred_element_type=jnp.float32)
                + jax.lax.dot(sel, src_lo, preferred_element_type=jnp.float32))

    g0 = gather(rankv == 1)
    fb = jnp.where(count > 0, g0, src[0:1])
    acc0 = mlp(fb - padcen)

    def slot(k, carry):
        acc, fb = carry
        gk = gather(rankv == k + 1)
        geff = jnp.where(count > k, gk, fb)
        h = mlp(geff - padcen)
        return (jnp.maximum(acc, h), fb)

    acc, _ = jax.lax.fori_loop(1, nsample, slot, (acc0, fb))
    out_ref[0] = acc


def _run_sa(src, xt, cen, layers, r2):
    b, n, C = src.shape
    s = cen.shape[1]
    h3 = layers[2][0].shape[1]
    wb = []
    specs = [
        pl.BlockSpec((1, n, C), lambda i: (i, 0, 0)),
        pl.BlockSpec((1, 3, n), lambda i: (i, 0, 0)),
        pl.BlockSpec((1, s, 3), lambda i: (i, 0, 0)),
    ]
    for (W, bb) in layers:
        wb += [W, bb.reshape(1, -1)]
        specs += [
            pl.BlockSpec(W.shape, lambda i: (0, 0)),
            pl.BlockSpec((1, bb.shape[0]), lambda i: (0, 0)),
        ]
    body = functools.partial(_sa_body, r2=r2, nsample=_NSAMPLE)
    return pl.pallas_call(
        body,
        grid=(b,),
        in_specs=specs,
        out_specs=pl.BlockSpec((1, s, h3), lambda i: (i, 0, 0)),
        out_shape=jax.ShapeDtypeStruct((b, s, h3), jnp.float32),
        compiler_params=_PAR,
    )(src, xt, cen, *wb)


def _saidx_body(cen_ref, xt_ref, out_ref, *, r2, nsample, n):
    b = pl.program_id(0)
    cen = cen_ref[0]
    xt = xt_ref[0]
    s = cen.shape[0]
    xr, yr, zr = xt[0:1], xt[1:2], xt[2:3]
    sum_b = (xr * xr + yr * yr) + zr * zr
    ca, cb, cc = cen[:, 0:1], cen[:, 1:2], cen[:, 2:3]
    sum_a = (ca * ca + cb * cb) + cc * cc
    cross = jnp.dot(cen, xt)
    sqd = (sum_a + sum_b) - 2.0 * cross
    valid = sqd <= r2
    rank = _prefix_sum_lanes(valid.astype(jnp.int32))
    count = rank[:, n - 1:n]
    rankv = jnp.where(valid, rank, 0)
    lane = _iota((s, n), 1)
    i0 = jnp.min(jnp.where(rankv == 1, lane, n), axis=1, keepdims=True)
    i0 = jnp.where(count > 0, i0, 0)
    base = b * n
    for k in range(nsample):
        ik = jnp.min(jnp.where(rankv == k + 1, lane, n), axis=1,
                     keepdims=True)
        ik = jnp.where(count > k, ik, i0)
        out_ref[0, :, k:k + 1] = ik + base


def _run_saidx(cen, xt, r2):
    b, s, _ = cen.shape
    n = xt.shape[2]
    body = functools.partial(_saidx_body, r2=r2, nsample=_NSAMPLE, n=n)
    return pl.pallas_call(
        body,
        grid=(b,),
        in_specs=[pl.BlockSpec((1, s, 3), lambda i: (i, 0, 0)),
                  pl.BlockSpec((1, 3, n), lambda i: (i, 0, 0))],
        out_specs=pl.BlockSpec((1, s, _NSAMPLE), lambda i: (i, 0, 0)),
        out_shape=jax.ShapeDtypeStruct((b, s, _NSAMPLE), jnp.int32),
    )(cen, xt)


def _sc_gather(table, idx):
    from jax.experimental.pallas import tpu_sc as plsc
    V, D = table.shape
    M = idx.shape[0]
    info = plsc.get_sparse_core_info()
    nw = info.num_cores * info.num_subcores
    b_per_w = M // nw
    chunk = b_per_w
    while chunk * D * 4 > 200_000:
        chunk //= 2
    n_chunks = b_per_w // chunk
    mesh = plsc.VectorSubcoreMesh(core_axis_name="c", subcore_axis_name="s")

    @functools.partial(
        pl.kernel, mesh=mesh,
        out_type=jax.ShapeDtypeStruct((M, D), jnp.float32),
        scratch_types=[
            pltpu.VMEM((chunk,), jnp.int32),
            pltpu.VMEM((chunk, D), jnp.float32),
            pltpu.SemaphoreType.DMA,
        ],
        compiler_params=pltpu.CompilerParams(use_tc_tiling_on_sc=False),
    )
    def k(table_hbm, idx_hbm, out_hbm, idx_v, rows_v, sem):
        wid = jax.lax.axis_index("s") * info.num_cores + jax.lax.axis_index("c")
        base = wid * b_per_w

        def body(ci, carry):
            off = base + ci * chunk
            pltpu.sync_copy(idx_hbm.at[pl.ds(off, chunk)], idx_v)
            pltpu.async_copy(table_hbm.at[idx_v], rows_v, sem).wait()
            pltpu.sync_copy(rows_v, out_hbm.at[pl.ds(off, chunk)])
            return carry

        jax.lax.fori_loop(0, n_chunks, body, 0)

    return k(table, idx)


def _samlp_body(g_ref, cen_ref, w1_ref, b1_ref, w2_ref, b2_ref,
                w3_ref, b3_ref, out_ref, *, nsample, cfeat):
    cen = cen_ref[0]
    s = cen.shape[0]
    padcen = jnp.concatenate(
        [cen, jnp.zeros((s, cfeat - 3), jnp.float32)], axis=1)
    w1, b1 = w1_ref[...], b1_ref[...]
    w2, b2 = w2_ref[...], b2_ref[...]
    w3, b3 = w3_ref[...], b3_ref[...]

    def mlp(x):
        h = jnp.maximum(jnp.dot(x, w1) + b1, 0.0)
        h = jnp.maximum(jnp.dot(h, w2) + b2, 0.0)
        return jnp.maximum(jnp.dot(h, w3) + b3, 0.0)

    def slot(k):
        blk = g_ref[0, pl.ds(k * s, s), :]
        return mlp(blk[:, :cfeat] - padcen)

    acc0 = slot(0)

    def body(k, acc):
        return jnp.maximum(acc, slot(k))

    out_ref[0] = jax.lax.fori_loop(1, nsample, body, acc0)


def _run_samlp(g3, cen, layers, cfeat):
    b, s, _ = cen.shape
    dpad = g3.shape[2]
    h3 = layers[2][0].shape[1]
    wb = []
    specs = [
        pl.BlockSpec((1, _NSAMPLE * s, dpad), lambda i: (i, 0, 0)),
        pl.BlockSpec((1, s, 3), lambda i: (i, 0, 0)),
    ]
    for (W, bb) in layers:
        wb += [W, bb.reshape(1, -1)]
        specs += [
            pl.BlockSpec(W.shape, lambda i: (0, 0)),
            pl.BlockSpec((1, bb.shape[0]), lambda i: (0, 0)),
        ]
    body = functools.partial(_samlp_body, nsample=_NSAMPLE, cfeat=cfeat)
    return pl.pallas_call(
        body,
        grid=(b,),
        in_specs=specs,
        out_specs=pl.BlockSpec((1, s, h3), lambda i: (i, 0, 0)),
        out_shape=jax.ShapeDtypeStruct((b, s, h3), jnp.float32),
    )(g3, cen, *wb)


def _run_sa_sc(src, xt, cen, layers, r2):
    b, n, C = src.shape
    cpad = (C + 15) // 16 * 16
    idx = _run_saidx(cen, xt, r2)
    idx_t = jnp.transpose(idx, (0, 2, 1)).reshape(-1)
    table = src.reshape(b * n, C)
    if cpad != C:
        table = jnp.pad(table, ((0, 0), (0, cpad - C)))
    gath = _sc_gather(table, idx_t)
    s = cen.shape[1]
    return _run_samlp(gath.reshape(b, _NSAMPLE * s, cpad), cen, layers, C)


def _three_nn_interp(qxyz, sum_a, xtk, fk):
    s = xtk.shape[1]
    q = qxyz.shape[0]
    lane_s = _iota((q, s), 1)
    xr, yr, zr = xtk[0:1], xtk[1:2], xtk[2:3]
    sum_b = (xr * xr + yr * yr) + zr * zr
    cross = jnp.dot(qxyz, xtk)
    d = (sum_a + sum_b) - 2.0 * cross

    fk_hi = fk.astype(jnp.bfloat16)
    fk_lo = (fk - fk_hi.astype(jnp.float32)).astype(jnp.bfloat16)
    rs, gs = [], []
    for _ in range(3):
        m = jnp.min(d, axis=1, keepdims=True)
        idx = jnp.min(jnp.where(d == m, lane_s, s), axis=1, keepdims=True)
        sel = (lane_s == idx).astype(jnp.bfloat16)
        gs.append(jax.lax.dot(sel, fk_hi, preferred_element_type=jnp.float32)
                  + jax.lax.dot(sel, fk_lo,
                                preferred_element_type=jnp.float32))
        dist = jnp.sqrt(jnp.maximum(m, 1e-12))
        rs.append(1.0 / (dist + 1e-8))
        d = jnp.where(lane_s == idx, 1e30, d)
    wsum = (rs[0] + rs[1]) + rs[2]
    w = [r / wsum for r in rs]
    return (gs[0] * w[0] + gs[1] * w[1]) + gs[2] * w[2]


def _head_body(xyz_ref, f1_ref,
               xt2_ref, f2_ref, wf0_ref, bf0_ref,
               xt3_ref, f3_ref, wf1_ref, bf1_ref,
               xt4_ref, f4_ref, wf2_ref, bf2_ref,
               wa1_ref, ba1_ref, wa2_ref, ba2_ref,
               wb1_ref, bb1_ref, wb2_ref, bb2_ref,
               wp1_ref, bp1_ref, wp2_ref, bp2_ref,
               out_ref, rf_ref):
    xyz = xyz_ref[0]
    f1 = f1_ref[0]
    q = xyz.shape[0]
    ax, ay, az = xyz[:, 0:1], xyz[:, 1:2], xyz[:, 2:3]
    sum_a = (ax * ax + ay * ay) + az * az

    ups = []
    for xtr, fr, wr, br in ((xt2_ref, f2_ref, wf0_ref, bf0_ref),
                            (xt3_ref, f3_ref, wf1_ref, bf1_ref),
                            (xt4_ref, f4_ref, wf2_ref, bf2_ref)):
        interp = _three_nn_interp(xyz, sum_a, xtr[0], fr[0])
        ups.append(jnp.maximum(jnp.dot(interp, wr[...]) + br[...], 0.0))

    fused = jnp.concatenate([xyz, f1] + ups, axis=1)

    fc_outs = []
    for w1r, b1r, w2r, b2r in ((wa1_ref, ba1_ref, wa2_ref, ba2_ref),
                               (wb1_ref, bb1_ref, wb2_ref, bb2_ref)):
        h = jnp.maximum(jnp.dot(fused, w1r[...]) + b1r[...], 0.0)
        fc_outs.append(jnp.maximum(jnp.dot(h, w2r[...]) + b2r[...], 0.0))
    rf = jnp.concatenate(fc_outs, axis=0)

    h = jnp.maximum(jnp.dot(rf, wp1_ref[...]) + bp1_ref[...], 0.0)
    out = jnp.dot(h, wp2_ref[...]) + bp2_ref[...]
    out_ref[0] = out
    rf_ref[0] = rf


def _run_head(xyz, f1, fp_ins, fc_layers, pcd_layers):
    b, q, _ = xyz.shape
    args = [xyz, f1]
    specs = [
        pl.BlockSpec((1, q, 3), lambda i: (i, 0, 0)),
        pl.BlockSpec((1, q, f1.shape[2]), lambda i: (i, 0, 0)),
    ]
    for xtk, fk, (W, bb) in fp_ins:
        s = xtk.shape[2]
        c = fk.shape[2]
        args += [xtk, fk, W, bb.reshape(1, -1)]
        specs += [
            pl.BlockSpec((1, 3, s), lambda i: (i, 0, 0)),
            pl.BlockSpec((1, s, c), lambda i: (i, 0, 0)),
            pl.BlockSpec(W.shape, lambda i: (0, 0)),
            pl.BlockSpec((1, bb.shape[0]), lambda i: (0, 0)),
        ]
    for layers in fc_layers + [pcd_layers]:
        for (W, bb) in layers:
            args += [W, bb.reshape(1, -1)]
            specs += [
                pl.BlockSpec(W.shape, lambda i: (0, 0)),
                pl.BlockSpec((1, bb.shape[0]), lambda i: (0, 0)),
            ]
    out_dim = pcd_layers[-1][0].shape[1]
    rf_dim = fc_layers[0][-1][0].shape[1]
    return pl.pallas_call(
        _head_body,
        grid=(b,),
        in_specs=specs,
        out_specs=[
            pl.BlockSpec((1, 2 * q, out_dim), lambda i: (i, 0, 0)),
            pl.BlockSpec((1, 2 * q, rf_dim), lambda i: (i, 0, 0)),
        ],
        out_shape=[
            jax.ShapeDtypeStruct((b, 2 * q, out_dim), jnp.float32),
            jax.ShapeDtypeStruct((b, 2 * q, rf_dim), jnp.float32),
        ],
        compiler_params=_PAR,
    )(*args)


def kernel(points, params):
    xyz = points[..., :3]
    feats0 = points[..., 3:]

    fps = _run_fps(xyz[..., 0], xyz[..., 1], xyz[..., 2])
    lvl_xyz = []
    lvl_xt = []
    for k in range(4):
        nx, ny, nz = fps[3 * k], fps[3 * k + 1], fps[3 * k + 2]
        lvl_xyz.append(jnp.stack([nx, ny, nz], axis=-1))
        lvl_xt.append(jnp.stack([nx, ny, nz], axis=1))

    src_xyz = xyz
    src_xt = jnp.transpose(xyz, (0, 2, 1))
    feats = feats0
    lvl_feats = []
    for k in range(4):
        src = jnp.concatenate([src_xyz, feats], axis=-1)
        r2 = float(_RADII[k]) * float(_RADII[k])
        run = _run_sa_sc if k == 1 else _run_sa
        feats = run(src, src_xt, lvl_xyz[k], params['sa'][k], r2)
        lvl_feats.append(feats)
        src_xyz = lvl_xyz[k]
        src_xt = lvl_xt[k]

    fp_ins = [
        (lvl_xt[1], lvl_feats[1], params['fp'][0][0]),
        (lvl_xt[2], lvl_feats[2], params['fp'][1][0]),
        (lvl_xt[3], lvl_feats[3], params['fp'][2][0]),
    ]
    out, rf = _run_head(xyz, lvl_feats[0], fp_ins,
                        params['fc'], params['pcd'])
    return out, rf


# speedup vs baseline: 1.0334x; 1.0334x over previous
import functools

import jax
import jax.numpy as jnp
from jax.experimental import pallas as pl
from jax.experimental.pallas import tpu as pltpu

_HIGH = jax.lax.Precision.HIGHEST
_PAR = pltpu.CompilerParams(dimension_semantics=("parallel",))

_NPOINTS = [1024, 512, 256, 128]
_RADII = [0.05, 0.1, 0.2, 0.3]
_NSAMPLE = 32


def _iota(shape, dim):
    return jax.lax.broadcasted_iota(jnp.int32, shape, dim)


def _fps_level(X, Y, Z, npoint):
    b, n = X.shape
    lane_n = _iota((b, n), 1)
    lane_p = _iota((b, npoint), 1)

    def body(i, st):
        dists, far, nx, ny, nz = st
        sel = lane_n == far
        cx = jnp.sum(jnp.where(sel, X, 0.0), axis=1, keepdims=True)
        cy = jnp.sum(jnp.where(sel, Y, 0.0), axis=1, keepdims=True)
        cz = jnp.sum(jnp.where(sel, Z, 0.0), axis=1, keepdims=True)
        m = lane_p == i
        nx = jnp.where(m, cx, nx)
        ny = jnp.where(m, cy, ny)
        nz = jnp.where(m, cz, nz)
        dx = X - cx
        dy = Y - cy
        dz = Z - cz
        d = (dx * dx + dz * dz) + dy * dy
        dists = jnp.minimum(dists, d)
        mx = jnp.max(dists, axis=1, keepdims=True)
        far = jnp.min(jnp.where(dists == mx, lane_n, n), axis=1, keepdims=True)
        return (dists, far, nx, ny, nz)

    init = (
        jnp.full((b, n), 1e10, jnp.float32),
        jnp.zeros((b, 1), jnp.int32),
        jnp.zeros((b, npoint), jnp.float32),
        jnp.zeros((b, npoint), jnp.float32),
        jnp.zeros((b, npoint), jnp.float32),
    )
    _, _, nx, ny, nz = jax.lax.fori_loop(0, npoint, body, init)
    return nx, ny, nz


def _fps_body(x_ref, y_ref, z_ref, *out_refs):
    X, Y, Z = x_ref[...], y_ref[...], z_ref[...]
    for lvl, npoint in enumerate(_NPOINTS):
        X, Y, Z = _fps_level(X, Y, Z, npoint)
        out_refs[3 * lvl][...] = X
        out_refs[3 * lvl + 1][...] = Y
        out_refs[3 * lvl + 2][...] = Z


def _run_fps(x0, y0, z0):
    b = x0.shape[0]
    outs = []
    for npoint in _NPOINTS:
        outs += [jax.ShapeDtypeStruct((b, npoint), jnp.float32)] * 3
    return pl.pallas_call(_fps_body, out_shape=outs)(x0, y0, z0)


def _prefix_sum_lanes(x):
    s, n = x.shape
    sh = 1
    while sh < n:
        pad = jnp.zeros((s, sh), x.dtype)
        x = x + jnp.concatenate([pad, x[:, : n - sh]], axis=1)
        sh *= 2
    return x


def _sa_body(src_ref, xt_ref, cen_ref, w1_ref, b1_ref, w2_ref, b2_ref,
             w3_ref, b3_ref, out_ref, *, r2, nsample):
    src = src_ref[0]
    xt = xt_ref[0]
    cen = cen_ref[0]
    n = src.shape[0]
    s = cen.shape[0]
    C = src.shape[1]

    xr, yr, zr = xt[0:1], xt[1:2], xt[2:3]
    sum_b = (xr * xr + yr * yr) + zr * zr
    ca, cb, cc = cen[:, 0:1], cen[:, 1:2], cen[:, 2:3]
    sum_a = (ca * ca + cb * cb) + cc * cc
    cross = jnp.dot(cen, xt)
    sqd = (sum_a + sum_b) - 2.0 * cross

    valid = sqd <= r2
    rank = _prefix_sum_lanes(valid.astype(jnp.int32))
    count = rank[:, n - 1:n]
    rankv = jnp.where(valid, rank, 0)
    padcen = jnp.concatenate(
        [cen, jnp.zeros((s, C - 3), jnp.float32)], axis=1)

    w1, b1 = w1_ref[...], b1_ref[...]
    w2, b2 = w2_ref[...], b2_ref[...]
    w3, b3 = w3_ref[...], b3_ref[...]

    def mlp(x):
        h = jnp.maximum(jnp.dot(x, w1) + b1, 0.0)
        h = jnp.maximum(jnp.dot(h, w2) + b2, 0.0)
        return jnp.maximum(jnp.dot(h, w3) + b3, 0.0)

    src_hi = src.astype(jnp.bfloat16)
    src_lo = (src - src_hi.astype(jnp.float32)).astype(jnp.bfloat16)

    def gather(sel_bool):
        sel = sel_bool.astype(jnp.bfloat16)
        return (jax.lax.dot(sel, src_hi, preferred_element_type=jnp.float32)
                + jax.lax.dot(sel, src_lo, preferred_element_type=jnp.float32))

    g0 = gather(rankv == 1)
    fb = jnp.where(count > 0, g0, src[0:1])
    acc0 = mlp(fb - padcen)

    def slot(k, carry):
        acc, fb = carry
        gk = gather(rankv == k + 1)
        geff = jnp.where(count > k, gk, fb)
        h = mlp(geff - padcen)
        return (jnp.maximum(acc, h), fb)

    acc, _ = jax.lax.fori_loop(1, nsample, slot, (acc0, fb))
    out_ref[0] = acc


def _run_sa(src, xt, cen, layers, r2):
    b, n, C = src.shape
    s = cen.shape[1]
    h3 = layers[2][0].shape[1]
    wb = []
    specs = [
        pl.BlockSpec((1, n, C), lambda i: (i, 0, 0)),
        pl.BlockSpec((1, 3, n), lambda i: (i, 0, 0)),
        pl.BlockSpec((1, s, 3), lambda i: (i, 0, 0)),
    ]
    for (W, bb) in layers:
        wb += [W, bb.reshape(1, -1)]
        specs += [
            pl.BlockSpec(W.shape, lambda i: (0, 0)),
            pl.BlockSpec((1, bb.shape[0]), lambda i: (0, 0)),
        ]
    body = functools.partial(_sa_body, r2=r2, nsample=_NSAMPLE)
    return pl.pallas_call(
        body,
        grid=(b,),
        in_specs=specs,
        out_specs=pl.BlockSpec((1, s, h3), lambda i: (i, 0, 0)),
        out_shape=jax.ShapeDtypeStruct((b, s, h3), jnp.float32),
        compiler_params=_PAR,
    )(src, xt, cen, *wb)


def _saidx_body(cen_ref, xt_ref, out_ref, *, r2, nsample, n):
    b = pl.program_id(0)
    cen = cen_ref[0]
    xt = xt_ref[0]
    s = cen.shape[0]
    xr, yr, zr = xt[0:1], xt[1:2], xt[2:3]
    sum_b = (xr * xr + yr * yr) + zr * zr
    ca, cb, cc = cen[:, 0:1], cen[:, 1:2], cen[:, 2:3]
    sum_a = (ca * ca + cb * cb) + cc * cc
    cross = jnp.dot(cen, xt)
    sqd = (sum_a + sum_b) - 2.0 * cross
    valid = sqd <= r2
    rank = _prefix_sum_lanes(valid.astype(jnp.int32))
    count = rank[:, n - 1:n]
    rankv = jnp.where(valid, rank, 0)
    lane = _iota((s, n), 1)
    i0 = jnp.min(jnp.where(rankv == 1, lane, n), axis=1, keepdims=True)
    i0 = jnp.where(count > 0, i0, 0)
    base = b * n
    for k in range(nsample):
        ik = jnp.min(jnp.where(rankv == k + 1, lane, n), axis=1,
                     keepdims=True)
        ik = jnp.where(count > k, ik, i0)
        out_ref[0, :, k:k + 1] = ik + base


def _run_saidx(cen, xt, r2):
    b, s, _ = cen.shape
    n = xt.shape[2]
    body = functools.partial(_saidx_body, r2=r2, nsample=_NSAMPLE, n=n)
    return pl.pallas_call(
        body,
        grid=(b,),
        in_specs=[pl.BlockSpec((1, s, 3), lambda i: (i, 0, 0)),
                  pl.BlockSpec((1, 3, n), lambda i: (i, 0, 0))],
        out_specs=pl.BlockSpec((1, s, _NSAMPLE), lambda i: (i, 0, 0)),
        out_shape=jax.ShapeDtypeStruct((b, s, _NSAMPLE), jnp.int32),
    )(cen, xt)


def _sc_gather(table, idx):
    from jax.experimental.pallas import tpu_sc as plsc
    V, D = table.shape
    M = idx.shape[0]
    info = plsc.get_sparse_core_info()
    nw = info.num_cores * info.num_subcores
    b_per_w = M // nw
    chunk = b_per_w
    while chunk * D * 4 > 200_000:
        chunk //= 2
    n_chunks = b_per_w // chunk
    mesh = plsc.VectorSubcoreMesh(core_axis_name="c", subcore_axis_name="s")

    @functools.partial(
        pl.kernel, mesh=mesh,
        out_type=jax.ShapeDtypeStruct((M, D), jnp.float32),
        scratch_types=[
            pltpu.VMEM((chunk,), jnp.int32),
            pltpu.VMEM((chunk, D), jnp.float32),
            pltpu.SemaphoreType.DMA,
        ],
        compiler_params=pltpu.CompilerParams(use_tc_tiling_on_sc=False),
    )
    def k(table_hbm, idx_hbm, out_hbm, idx_v, rows_v, sem):
        wid = jax.lax.axis_index("s") * info.num_cores + jax.lax.axis_index("c")
        base = wid * b_per_w

        def body(ci, carry):
            off = base + ci * chunk
            pltpu.sync_copy(idx_hbm.at[pl.ds(off, chunk)], idx_v)
            pltpu.async_copy(table_hbm.at[idx_v], rows_v, sem).wait()
            pltpu.sync_copy(rows_v, out_hbm.at[pl.ds(off, chunk)])
            return carry

        jax.lax.fori_loop(0, n_chunks, body, 0)

    return k(table, idx)


def _samlp_body(g_ref, cen_ref, w1_ref, b1_ref, w2_ref, b2_ref,
                w3_ref, b3_ref, out_ref, *, nsample, cfeat):
    cen = cen_ref[0]
    s = cen.shape[0]
    padcen = jnp.concatenate(
        [cen, jnp.zeros((s, cfeat - 3), jnp.float32)], axis=1)
    w1, b1 = w1_ref[...], b1_ref[...]
    w2, b2 = w2_ref[...], b2_ref[...]
    w3, b3 = w3_ref[...], b3_ref[...]

    def mlp(x):
        h = jnp.maximum(jnp.dot(x, w1) + b1, 0.0)
        h = jnp.maximum(jnp.dot(h, w2) + b2, 0.0)
        return jnp.maximum(jnp.dot(h, w3) + b3, 0.0)

    def slot(k):
        blk = g_ref[0, pl.ds(k * s, s), :]
        return mlp(blk[:, :cfeat] - padcen)

    acc0 = slot(0)

    def body(k, acc):
        return jnp.maximum(acc, slot(k))

    out_ref[0] = jax.lax.fori_loop(1, nsample, body, acc0)


def _run_samlp(g3, cen, layers, cfeat):
    b, s, _ = cen.shape
    dpad = g3.shape[2]
    h3 = layers[2][0].shape[1]
    wb = []
    specs = [
        pl.BlockSpec((1, _NSAMPLE * s, dpad), lambda i: (i, 0, 0)),
        pl.BlockSpec((1, s, 3), lambda i: (i, 0, 0)),
    ]
    for (W, bb) in layers:
        wb += [W, bb.reshape(1, -1)]
        specs += [
            pl.BlockSpec(W.shape, lambda i: (0, 0)),
            pl.BlockSpec((1, bb.shape[0]), lambda i: (0, 0)),
        ]
    body = functools.partial(_samlp_body, nsample=_NSAMPLE, cfeat=cfeat)
    return pl.pallas_call(
        body,
        grid=(b,),
        in_specs=specs,
        out_specs=pl.BlockSpec((1, s, h3), lambda i: (i, 0, 0)),
        out_shape=jax.ShapeDtypeStruct((b, s, h3), jnp.float32),
    )(g3, cen, *wb)


def _run_sa_sc(src, xt, cen, layers, r2):
    b, n, C = src.shape
    cpad = (C + 15) // 16 * 16
    idx = _run_saidx(cen, xt, r2)
    idx_t = jnp.transpose(idx, (0, 2, 1)).reshape(-1)
    table = src.reshape(b * n, C)
    if cpad != C:
        table = jnp.pad(table, ((0, 0), (0, cpad - C)))
    gath = _sc_gather(table, idx_t)
    s = cen.shape[1]
    return _run_samlp(gath.reshape(b, _NSAMPLE * s, cpad), cen, layers, C)


def _three_nn_interp(qxyz, sum_a, xtk, fk):
    s = xtk.shape[1]
    q = qxyz.shape[0]
    lane_s = _iota((q, s), 1)
    xr, yr, zr = xtk[0:1], xtk[1:2], xtk[2:3]
    sum_b = (xr * xr + yr * yr) + zr * zr
    cross = jnp.dot(qxyz, xtk)
    d = (sum_a + sum_b) - 2.0 * cross

    fk_hi = fk.astype(jnp.bfloat16)
    fk_lo = (fk - fk_hi.astype(jnp.float32)).astype(jnp.bfloat16)
    rs, gs = [], []
    for _ in range(3):
        m = jnp.min(d, axis=1, keepdims=True)
        idx = jnp.min(jnp.where(d == m, lane_s, s), axis=1, keepdims=True)
        sel = (lane_s == idx).astype(jnp.bfloat16)
        gs.append(jax.lax.dot(sel, fk_hi, preferred_element_type=jnp.float32)
                  + jax.lax.dot(sel, fk_lo,
                                preferred_element_type=jnp.float32))
        dist = jnp.sqrt(jnp.maximum(m, 1e-12))
        rs.append(1.0 / (dist + 1e-8))
        d = jnp.where(lane_s == idx, 1e30, d)
    wsum = (rs[0] + rs[1]) + rs[2]
    w = [r / wsum for r in rs]
    return (gs[0] * w[0] + gs[1] * w[1]) + gs[2] * w[2]


def _head_body(xyz_ref, f1_ref,
               xt2_ref, f2_ref, wf0_ref, bf0_ref,
               xt3_ref, f3_ref, wf1_ref, bf1_ref,
               xt4_ref, f4_ref, wf2_ref, bf2_ref,
               wa1_ref, ba1_ref, wa2_ref, ba2_ref,
               wb1_ref, bb1_ref, wb2_ref, bb2_ref,
               wp1_ref, bp1_ref, wp2_ref, bp2_ref,
               out_ref, rf_ref):
    xyz = xyz_ref[0]
    f1 = f1_ref[0]
    q = xyz.shape[0]
    ax, ay, az = xyz[:, 0:1], xyz[:, 1:2], xyz[:, 2:3]
    sum_a = (ax * ax + ay * ay) + az * az

    ups = []
    for xtr, fr, wr, br in ((xt2_ref, f2_ref, wf0_ref, bf0_ref),
                            (xt3_ref, f3_ref, wf1_ref, bf1_ref),
                            (xt4_ref, f4_ref, wf2_ref, bf2_ref)):
        interp = _three_nn_interp(xyz, sum_a, xtr[0], fr[0])
        ups.append(jnp.maximum(jnp.dot(interp, wr[...]) + br[...], 0.0))

    fused = jnp.concatenate([xyz, f1] + ups, axis=1)

    fc_outs = []
    for w1r, b1r, w2r, b2r in ((wa1_ref, ba1_ref, wa2_ref, ba2_ref),
                               (wb1_ref, bb1_ref, wb2_ref, bb2_ref)):
        h = jnp.maximum(jnp.dot(fused, w1r[...]) + b1r[...], 0.0)
        fc_outs.append(jnp.maximum(jnp.dot(h, w2r[...]) + b2r[...], 0.0))
    rf = jnp.concatenate(fc_outs, axis=0)

    h = jnp.maximum(jnp.dot(rf, wp1_ref[...]) + bp1_ref[...], 0.0)
    out = jnp.dot(h, wp2_ref[...]) + bp2_ref[...]
    out_ref[0] = out
    rf_ref[0] = rf


def _run_head(xyz, f1, fp_ins, fc_layers, pcd_layers):
    b, q, _ = xyz.shape
    args = [xyz, f1]
    specs = [
        pl.BlockSpec((1, q, 3), lambda i: (i, 0, 0)),
        pl.BlockSpec((1, q, f1.shape[2]), lambda i: (i, 0, 0)),
    ]
    for xtk, fk, (W, bb) in fp_ins:
        s = xtk.shape[2]
        c = fk.shape[2]
        args += [xtk, fk, W, bb.reshape(1, -1)]
        specs += [
            pl.BlockSpec((1, 3, s), lambda i: (i, 0, 0)),
            pl.BlockSpec((1, s, c), lambda i: (i, 0, 0)),
            pl.BlockSpec(W.shape, lambda i: (0, 0)),
            pl.BlockSpec((1, bb.shape[0]), lambda i: (0, 0)),
        ]
    for layers in fc_layers + [pcd_layers]:
        for (W, bb) in layers:
            args += [W, bb.reshape(1, -1)]
            specs += [
                pl.BlockSpec(W.shape, lambda i: (0, 0)),
                pl.BlockSpec((1, bb.shape[0]), lambda i: (0, 0)),
            ]
    out_dim = pcd_layers[-1][0].shape[1]
    rf_dim = fc_layers[0][-1][0].shape[1]
    return pl.pallas_call(
        _head_body,
        grid=(b,),
        in_specs=specs,
        out_specs=[
            pl.BlockSpec((1, 2 * q, out_dim), lambda i: (i, 0, 0)),
            pl.BlockSpec((1, 2 * q, rf_dim), lambda i: (i, 0, 0)),
        ],
        out_shape=[
            jax.ShapeDtypeStruct((b, 2 * q, out_dim), jnp.float32),
            jax.ShapeDtypeStruct((b, 2 * q, rf_dim), jnp.float32),
        ],
        compiler_params=_PAR,
    )(*args)


def kernel(points, params):
    xyz = points[..., :3]
    feats0 = points[..., 3:]

    fps = _run_fps(xyz[..., 0], xyz[..., 1], xyz[..., 2])
    lvl_xyz = []
    lvl_xt = []
    for k in range(4):
        nx, ny, nz = fps[3 * k], fps[3 * k + 1], fps[3 * k + 2]
        lvl_xyz.append(jnp.stack([nx, ny, nz], axis=-1))
        lvl_xt.append(jnp.stack([nx, ny, nz], axis=1))

    src_xyz = xyz
    src_xt = jnp.transpose(xyz, (0, 2, 1))
    feats = feats0
    lvl_feats = []
    for k in range(4):
        src = jnp.concatenate([src_xyz, feats], axis=-1)
        r2 = float(_RADII[k]) * float(_RADII[k])
        run = _run_sa_sc if k == 1 else _run_sa
        feats = run(src, src_xt, lvl_xyz[k], params['sa'][k], r2)
        lvl_feats.append(feats)
        src_xyz = lvl_xyz[k]
        src_xt = lvl_xt[k]

    fp_ins = [
        (lvl_xt[1], lvl_feats[1], params['fp'][0][0]),
        (lvl_xt[2], lvl_feats[2], params['fp'][1][0]),
        (lvl_xt[3], lvl_feats[3], params['fp'][2][0]),
    ]
    out, rf = _run_head(xyz, lvl_feats[0], fp_ins,
                        params['fc'], params['pcd'])
    return out, rf
